# baseline (speedup 1.0000x reference)
import functools

import jax
import jax.numpy as jnp
from jax import lax
from jax.experimental import pallas as pl
from jax.experimental.pallas import tpu as pltpu
from jax.experimental.pallas import tpu_sc as plsc

D = 64
NC = 2
NS = 16
NW = NC * NS
LANES = 16


def _proj_body(x_ref, wt_ref, o_ref):
    o_ref[...] = jnp.dot(x_ref[...], wt_ref[...],
                         preferred_element_type=jnp.float32)


def _projb_body(x_ref, wt_ref, b_ref, o_ref):
    o_ref[...] = jnp.dot(x_ref[...], wt_ref[...],
                         preferred_element_type=jnp.float32) + b_ref[...]


def _project_tables(char_table, lang_table, W, b):
    V, _ = char_table.shape
    Vl, _ = lang_table.shape
    wt_c = W[:, :D].T
    wt_l = W[:, D:].T
    blk = 10000
    char_proj = pl.pallas_call(
        _proj_body,
        grid=(V // blk,),
        in_specs=[pl.BlockSpec((blk, D), lambda i: (i, 0)),
                  pl.BlockSpec((D, D), lambda i: (0, 0))],
        out_specs=pl.BlockSpec((blk, D), lambda i: (i, 0)),
        out_shape=jax.ShapeDtypeStruct((V, D), jnp.float32),
    )(char_table, wt_c)
    lang_proj = pl.pallas_call(
        _projb_body,
        in_specs=[pl.BlockSpec((Vl, D), lambda: (0, 0)),
                  pl.BlockSpec((D, D), lambda: (0, 0)),
                  pl.BlockSpec((1, D), lambda: (0, 0))],
        out_specs=pl.BlockSpec((Vl, D), lambda: (0, 0)),
        out_shape=jax.ShapeDtypeStruct((Vl, D), jnp.float32),
    )(lang_table, wt_l, b.reshape(1, D))
    return char_proj, lang_proj


def _make_gather_add(n_tokens, chunk):
    per_w = n_tokens // NW
    n_chunks = per_w // chunk
    mesh = plsc.VectorSubcoreMesh(core_axis_name="c", subcore_axis_name="s")

    @functools.partial(
        pl.kernel,
        out_type=jax.ShapeDtypeStruct((n_tokens, D), jnp.float32),
        mesh=mesh,
        compiler_params=pltpu.CompilerParams(use_tc_tiling_on_sc=False),
        scratch_types=[
            pltpu.VMEM((n_chunks, chunk), jnp.int32),
            pltpu.VMEM((n_chunks, chunk), jnp.int32),
            pltpu.VMEM((2, chunk, D), jnp.float32),
            pltpu.VMEM((2, chunk, D), jnp.float32),
            pltpu.VMEM((2, chunk, D), jnp.float32),
            pltpu.SemaphoreType.DMA,
            pltpu.SemaphoreType.DMA,
            pltpu.SemaphoreType.DMA,
            pltpu.SemaphoreType.DMA,
            pltpu.SemaphoreType.DMA,
            pltpu.SemaphoreType.DMA,
        ],
    )
    def k(char_hbm, lang_hbm, ci_hbm, li_hbm, out_hbm,
          ci_all, li_all, buf_c, buf_l, buf_o,
          gc0, gc1, gl0, gl1, ss0, ss1):
        gsem_c = [gc0, gc1]
        gsem_l = [gl0, gl1]
        ssem = [ss0, ss1]
        wid = lax.axis_index("s") * NC + lax.axis_index("c")
        cbase = wid * n_chunks
        tbase = cbase * chunk

        pltpu.sync_copy(ci_hbm.at[pl.ds(cbase, n_chunks)], ci_all)
        pltpu.sync_copy(li_hbm.at[pl.ds(cbase, n_chunks)], li_all)

        def start_gather(j, b):
            pltpu.async_copy(char_hbm.at[ci_all.at[j]], buf_c.at[b], gsem_c[b])
            pltpu.async_copy(lang_hbm.at[li_all.at[j]], buf_l.at[b], gsem_l[b])

        def wait_gather(j, b):
            pltpu.make_async_copy(char_hbm.at[ci_all.at[j]],
                                  buf_c.at[b], gsem_c[b]).wait()
            pltpu.make_async_copy(lang_hbm.at[li_all.at[j]],
                                  buf_l.at[b], gsem_l[b]).wait()

        def start_store(j, b):
            pltpu.async_copy(buf_o.at[b],
                             out_hbm.at[pl.ds(tbase + j * chunk, chunk)],
                             ssem[b])

        def wait_store(j, b):
            pltpu.make_async_copy(buf_o.at[b],
                                  out_hbm.at[pl.ds(tbase + j * chunk, chunk)],
                                  ssem[b]).wait()

        start_gather(0, 0)
        start_gather(1, 1)

        def body(g, carry):
            for b in range(2):
                j = g * 2 + b
                wait_gather(j, b)

                @pl.when(j >= 2)
                def _():
                    wait_store(j - 2, b)

                def add_row(r, carry2):
                    for p in range(D // LANES):
                        sl = pl.ds(p * LANES, LANES)
                        buf_o[b, r, sl] = buf_c[b, r, sl] + buf_l[b, r, sl]
                    return carry2

                lax.fori_loop(0, chunk, add_row, 0, unroll=2)
                start_store(j, b)

                @pl.when(j + 2 < n_chunks)
                def _():
                    start_gather(j + 2, b)

            return carry

        lax.fori_loop(0, n_chunks // 2, body, 0)
        wait_store(n_chunks - 2, 0)
        wait_store(n_chunks - 1, 1)

    return k


def _fin_body(x_ref, i_ref, o_ref):
    o_ref[...] = jax.lax.dot_general(
        x_ref[...], i_ref[...], (((0,), (0,)), ((), ())),
        preferred_element_type=jnp.float32)


def _finalize(out_flat2, B, L):
    bb = 256
    cols = L * D
    return pl.pallas_call(
        _fin_body,
        grid=(B // bb,),
        in_specs=[pl.BlockSpec((bb, cols), lambda i: (i, 0)),
                  pl.BlockSpec((bb, bb), lambda i: (0, 0))],
        out_specs=pl.BlockSpec((cols, bb), lambda i: (0, i)),
        out_shape=jax.ShapeDtypeStruct((cols, B), jnp.float32),
    )(out_flat2, jnp.eye(bb, dtype=jnp.float32))


def kernel(char_indices, lang_indices, char_table, lang_table, W, b):
    B, L = char_indices.shape
    n = B * L
    char_proj, lang_proj = _project_tables(char_table, lang_table, W, b)
    chunk = 128
    ci = char_indices.reshape(n // chunk, chunk)
    li = lang_indices.reshape(n // chunk, chunk)
    gather_add = _make_gather_add(n, chunk=chunk)
    out = gather_add(char_proj, lang_proj, ci, li)
    out_t = _finalize(out.reshape(B, L * D), B, L)
    return jnp.transpose(out_t.reshape(L, D, B), (2, 0, 1))


# speedup vs baseline: 1.0010x; 1.0010x over previous
import functools

import jax
import jax.numpy as jnp
from jax import lax
from jax.experimental import pallas as pl
from jax.experimental.pallas import tpu as pltpu
from jax.experimental.pallas import tpu_sc as plsc

D = 64
NC = 2
NS = 16
NW = NC * NS
LANES = 16


def _proj_body(x_ref, wt_ref, o_ref):
    o_ref[...] = jnp.dot(x_ref[...], wt_ref[...],
                         preferred_element_type=jnp.float32)


def _projb_body(x_ref, wt_ref, b_ref, o_ref):
    o_ref[...] = jnp.dot(x_ref[...], wt_ref[...],
                         preferred_element_type=jnp.float32) + b_ref[...]


def _project_tables(char_table, lang_table, W, b):
    V, _ = char_table.shape
    Vl, _ = lang_table.shape
    wt_c = W[:, :D].T
    wt_l = W[:, D:].T
    blk = 10000
    char_proj = pl.pallas_call(
        _proj_body,
        grid=(V // blk,),
        in_specs=[pl.BlockSpec((blk, D), lambda i: (i, 0)),
                  pl.BlockSpec((D, D), lambda i: (0, 0))],
        out_specs=pl.BlockSpec((blk, D), lambda i: (i, 0)),
        out_shape=jax.ShapeDtypeStruct((V, D), jnp.float32),
    )(char_table, wt_c)
    lang_proj = pl.pallas_call(
        _projb_body,
        in_specs=[pl.BlockSpec((Vl, D), lambda: (0, 0)),
                  pl.BlockSpec((D, D), lambda: (0, 0)),
                  pl.BlockSpec((1, D), lambda: (0, 0))],
        out_specs=pl.BlockSpec((Vl, D), lambda: (0, 0)),
        out_shape=jax.ShapeDtypeStruct((Vl, D), jnp.float32),
    )(lang_table, wt_l, b.reshape(1, D))
    return char_proj, lang_proj


def _make_gather_add(n_tokens, chunk):
    per_w = n_tokens // NW
    n_chunks = per_w // chunk
    mesh = plsc.VectorSubcoreMesh(core_axis_name="c", subcore_axis_name="s")

    @functools.partial(
        pl.kernel,
        out_type=jax.ShapeDtypeStruct((n_tokens, D), jnp.float32),
        mesh=mesh,
        compiler_params=pltpu.CompilerParams(use_tc_tiling_on_sc=False),
        scratch_types=[
            pltpu.VMEM((n_chunks, chunk), jnp.int32),
            pltpu.VMEM((n_chunks, chunk), jnp.int32),
            pltpu.VMEM((2, chunk, D), jnp.float32),
            pltpu.VMEM((2, chunk, D), jnp.float32),
            pltpu.VMEM((2, chunk, D), jnp.float32),
            pltpu.VMEM_SHARED((1000, D), jnp.float32),
            pltpu.SemaphoreType.DMA,
            pltpu.SemaphoreType.DMA,
            pltpu.SemaphoreType.DMA,
            pltpu.SemaphoreType.DMA,
            pltpu.SemaphoreType.DMA,
            pltpu.SemaphoreType.DMA,
        ],
    )
    def k(char_hbm, lang_hbm, ci_hbm, li_hbm, out_hbm,
          ci_all, li_all, buf_c, buf_l, buf_o, lang_v,
          gc0, gc1, gl0, gl1, ss0, ss1):
        gsem_c = [gc0, gc1]
        gsem_l = [gl0, gl1]
        ssem = [ss0, ss1]
        wid = lax.axis_index("s") * NC + lax.axis_index("c")
        cbase = wid * n_chunks
        tbase = cbase * chunk

        pltpu.sync_copy(ci_hbm.at[pl.ds(cbase, n_chunks)], ci_all)
        pltpu.sync_copy(li_hbm.at[pl.ds(cbase, n_chunks)], li_all)
        @pl.when(lax.axis_index("s") == 0)
        def _():
            pltpu.sync_copy(lang_hbm, lang_v)

        plsc.subcore_barrier()

        def start_gather(j, b):
            pltpu.async_copy(char_hbm.at[ci_all.at[j]], buf_c.at[b], gsem_c[b])
            pltpu.async_copy(lang_v.at[li_all.at[j]], buf_l.at[b], gsem_l[b])

        def wait_gather(j, b):
            pltpu.make_async_copy(char_hbm.at[ci_all.at[j]],
                                  buf_c.at[b], gsem_c[b]).wait()
            pltpu.make_async_copy(lang_v.at[li_all.at[j]],
                                  buf_l.at[b], gsem_l[b]).wait()

        def start_store(j, b):
            pltpu.async_copy(buf_o.at[b],
                             out_hbm.at[pl.ds(tbase + j * chunk, chunk)],
                             ssem[b])

        def wait_store(j, b):
            pltpu.make_async_copy(buf_o.at[b],
                                  out_hbm.at[pl.ds(tbase + j * chunk, chunk)],
                                  ssem[b]).wait()

        start_gather(0, 0)
        start_gather(1, 1)

        def body(g, carry):
            for b in range(2):
                j = g * 2 + b
                wait_gather(j, b)

                @pl.when(j >= 2)
                def _():
                    wait_store(j - 2, b)

                def add_row(r, carry2):
                    for p in range(D // LANES):
                        sl = pl.ds(p * LANES, LANES)
                        buf_o[b, r, sl] = buf_c[b, r, sl] + buf_l[b, r, sl]
                    return carry2

                lax.fori_loop(0, chunk, add_row, 0, unroll=2)
                start_store(j, b)

                @pl.when(j + 2 < n_chunks)
                def _():
                    start_gather(j + 2, b)

            return carry

        lax.fori_loop(0, n_chunks // 2, body, 0)
        wait_store(n_chunks - 2, 0)
        wait_store(n_chunks - 1, 1)

    return k


def _fin_body(x_ref, o_ref):
    o_ref[...] = jnp.swapaxes(x_ref[...], 0, 1)


def _finalize(out_flat2, B, L):
    bb = 256
    cols = L * D
    return pl.pallas_call(
        _fin_body,
        grid=(B // bb,),
        in_specs=[pl.BlockSpec((bb, cols), lambda i: (i, 0))],
        out_specs=pl.BlockSpec((cols, bb), lambda i: (0, i)),
        out_shape=jax.ShapeDtypeStruct((cols, B), jnp.float32),
    )(out_flat2)


def kernel(char_indices, lang_indices, char_table, lang_table, W, b):
    B, L = char_indices.shape
    n = B * L
    char_proj, lang_proj = _project_tables(char_table, lang_table, W, b)
    chunk = 128
    ci = char_indices.reshape(n // chunk, chunk)
    li = lang_indices.reshape(n // chunk, chunk)
    gather_add = _make_gather_add(n, chunk=chunk)
    out = gather_add(char_proj, lang_proj, ci, li)
    out_t = _finalize(out.reshape(B, L * D), B, L)
    return jnp.transpose(out_t.reshape(L, D, B), (2, 0, 1))


# speedup vs baseline: 1.1477x; 1.1466x over previous
import functools

import jax
import jax.numpy as jnp
from jax import lax
from jax.experimental import pallas as pl
from jax.experimental.pallas import tpu as pltpu
from jax.experimental.pallas import tpu_sc as plsc

D = 64
NC = 2
NS = 16
NW = NC * NS
LANES = 16


def _proj_body(x_ref, wt_ref, o_ref):
    o_ref[...] = jnp.dot(x_ref[...], wt_ref[...],
                         preferred_element_type=jnp.float32)


def _projb_body(x_ref, wt_ref, b_ref, o_ref):
    o_ref[...] = jnp.dot(x_ref[...], wt_ref[...],
                         preferred_element_type=jnp.float32) + b_ref[...]


def _project_tables(char_table, lang_table, W, b):
    V, _ = char_table.shape
    Vl, _ = lang_table.shape
    wt_c = W[:, :D].T
    wt_l = W[:, D:].T
    blk = 10000
    char_proj = pl.pallas_call(
        _proj_body,
        grid=(V // blk,),
        in_specs=[pl.BlockSpec((blk, D), lambda i: (i, 0)),
                  pl.BlockSpec((D, D), lambda i: (0, 0))],
        out_specs=pl.BlockSpec((blk, D), lambda i: (i, 0)),
        out_shape=jax.ShapeDtypeStruct((V, D), jnp.float32),
    )(char_table, wt_c)
    lang_proj = pl.pallas_call(
        _projb_body,
        in_specs=[pl.BlockSpec((Vl, D), lambda: (0, 0)),
                  pl.BlockSpec((D, D), lambda: (0, 0)),
                  pl.BlockSpec((1, D), lambda: (0, 0))],
        out_specs=pl.BlockSpec((Vl, D), lambda: (0, 0)),
        out_shape=jax.ShapeDtypeStruct((Vl, D), jnp.float32),
    )(lang_table, wt_l, b.reshape(1, D))
    return char_proj, lang_proj


def _make_gather_add(n_tokens, chunk):
    per_w = n_tokens // NW
    n_chunks = per_w // chunk
    mesh = plsc.VectorSubcoreMesh(core_axis_name="c", subcore_axis_name="s")

    @functools.partial(
        pl.kernel,
        out_type=jax.ShapeDtypeStruct((n_tokens, D), jnp.float32),
        mesh=mesh,
        compiler_params=pltpu.CompilerParams(use_tc_tiling_on_sc=False),
        scratch_types=[
            pltpu.VMEM((n_chunks, chunk), jnp.int32),
            pltpu.VMEM((n_chunks, chunk), jnp.int32),
            pltpu.VMEM((2, chunk, D), jnp.float32),
            pltpu.VMEM((2, chunk, D), jnp.float32),
            pltpu.VMEM((2, chunk, D), jnp.float32),
            pltpu.VMEM_SHARED((1000, D), jnp.float32),
            pltpu.SemaphoreType.DMA,
            pltpu.SemaphoreType.DMA,
            pltpu.SemaphoreType.DMA,
            pltpu.SemaphoreType.DMA,
            pltpu.SemaphoreType.DMA,
            pltpu.SemaphoreType.DMA,
        ],
    )
    def k(char_hbm, lang_hbm, ci_hbm, li_hbm, out_hbm,
          ci_all, li_all, buf_c, buf_l, buf_o, lang_v,
          gc0, gc1, gl0, gl1, ss0, ss1):
        gsem_c = [gc0, gc1]
        gsem_l = [gl0, gl1]
        ssem = [ss0, ss1]
        wid = lax.axis_index("s") * NC + lax.axis_index("c")
        cbase = wid * n_chunks
        tbase = cbase * chunk

        pltpu.sync_copy(ci_hbm.at[pl.ds(cbase, n_chunks)], ci_all)
        pltpu.sync_copy(li_hbm.at[pl.ds(cbase, n_chunks)], li_all)
        @pl.when(lax.axis_index("s") == 0)
        def _():
            pltpu.sync_copy(lang_hbm, lang_v)

        plsc.subcore_barrier()

        def start_gather(j, b):
            pltpu.async_copy(char_hbm.at[ci_all.at[j]], buf_c.at[b], gsem_c[b])
            pltpu.async_copy(lang_v.at[li_all.at[j]], buf_l.at[b], gsem_l[b])

        def wait_gather(j, b):
            pltpu.make_async_copy(char_hbm.at[ci_all.at[j]],
                                  buf_c.at[b], gsem_c[b]).wait()
            pltpu.make_async_copy(lang_v.at[li_all.at[j]],
                                  buf_l.at[b], gsem_l[b]).wait()

        def start_store(j, b):
            pltpu.async_copy(buf_o.at[b],
                             out_hbm.at[pl.ds(tbase + j * chunk, chunk)],
                             ssem[b])

        def wait_store(j, b):
            pltpu.make_async_copy(buf_o.at[b],
                                  out_hbm.at[pl.ds(tbase + j * chunk, chunk)],
                                  ssem[b]).wait()

        start_gather(0, 0)
        start_gather(1, 1)

        def body(g, carry):
            for b in range(2):
                j = g * 2 + b
                wait_gather(j, b)

                @pl.when(j >= 2)
                def _():
                    wait_store(j - 2, b)

                def add_row(r, carry2):
                    for p in range(D // LANES):
                        sl = pl.ds(p * LANES, LANES)
                        buf_o[b, r, sl] = buf_c[b, r, sl] + buf_l[b, r, sl]
                    return carry2

                lax.fori_loop(0, chunk, add_row, 0, unroll=2)
                start_store(j, b)

                @pl.when(j + 2 < n_chunks)
                def _():
                    start_gather(j + 2, b)

            return carry

        lax.fori_loop(0, n_chunks // 2, body, 0)
        wait_store(n_chunks - 2, 0)
        wait_store(n_chunks - 1, 1)

    return k


def _fin_body(x_ref, i_ref, o_ref):
    cols, bb = o_ref.shape
    nq = cols // (2 * D)
    x3 = x_ref[...].reshape(bb, nq, 2 * D)
    ident = i_ref[...]
    for q in range(nq):
        oq = jax.lax.dot_general(
            x3[:, q, :], ident, (((0,), (0,)), ((), ())),
            preferred_element_type=jnp.float32)
        o_ref[pl.ds(q * 2 * D, 2 * D), :] = oq


def _finalize(out_pair, B, L):
    bb = 256
    cols = L * D
    rows = bb * L // 2
    return pl.pallas_call(
        _fin_body,
        grid=(B // bb,),
        in_specs=[pl.BlockSpec((rows, 2 * D), lambda i: (i, 0)),
                  pl.BlockSpec((bb, bb), lambda i: (0, 0))],
        out_specs=pl.BlockSpec((cols, bb), lambda i: (0, i)),
        out_shape=jax.ShapeDtypeStruct((cols, B), jnp.float32),
    )(out_pair, jnp.eye(bb, dtype=jnp.float32))


def kernel(char_indices, lang_indices, char_table, lang_table, W, b):
    B, L = char_indices.shape
    n = B * L
    char_proj, lang_proj = _project_tables(char_table, lang_table, W, b)
    chunk = 128
    ci = char_indices.reshape(n // chunk, chunk)
    li = lang_indices.reshape(n // chunk, chunk)
    gather_add = _make_gather_add(n, chunk=chunk)
    out = gather_add(char_proj, lang_proj, ci, li)
    out_t = _finalize(out.reshape(n // 2, 2 * D), B, L)
    return jnp.transpose(out_t.reshape(L, D, B), (2, 0, 1))


# speedup vs baseline: 1.8634x; 1.6235x over previous
import functools

import jax
import jax.numpy as jnp
from jax import lax
from jax.experimental import pallas as pl
from jax.experimental.pallas import tpu as pltpu
from jax.experimental.pallas import tpu_sc as plsc

D = 64
NC = 2
NS = 16
NW = NC * NS
LANES = 16


def _proj_body(x_ref, wt_ref, o_ref):
    o_ref[...] = jnp.dot(x_ref[...], wt_ref[...],
                         preferred_element_type=jnp.float32)


def _projb_body(x_ref, wt_ref, b_ref, o_ref):
    o_ref[...] = jnp.dot(x_ref[...], wt_ref[...],
                         preferred_element_type=jnp.float32) + b_ref[...]


def _project_tables(char_table, lang_table, W, b):
    V, _ = char_table.shape
    Vl, _ = lang_table.shape
    wt_c = W[:, :D].T
    wt_l = W[:, D:].T
    blk = 10000
    char_proj = pl.pallas_call(
        _proj_body,
        grid=(V // blk,),
        in_specs=[pl.BlockSpec((blk, D), lambda i: (i, 0)),
                  pl.BlockSpec((D, D), lambda i: (0, 0))],
        out_specs=pl.BlockSpec((blk, D), lambda i: (i, 0)),
        out_shape=jax.ShapeDtypeStruct((V, D), jnp.float32),
    )(char_table, wt_c)
    lang_proj = pl.pallas_call(
        _projb_body,
        in_specs=[pl.BlockSpec((Vl, D), lambda: (0, 0)),
                  pl.BlockSpec((D, D), lambda: (0, 0)),
                  pl.BlockSpec((1, D), lambda: (0, 0))],
        out_specs=pl.BlockSpec((Vl, D), lambda: (0, 0)),
        out_shape=jax.ShapeDtypeStruct((Vl, D), jnp.float32),
    )(lang_table, wt_l, b.reshape(1, D))
    return char_proj, lang_proj


def _make_gather_add(n_tokens, chunk):
    per_w = n_tokens // NW
    n_chunks = per_w // chunk
    mesh = plsc.VectorSubcoreMesh(core_axis_name="c", subcore_axis_name="s")

    @functools.partial(
        pl.kernel,
        out_type=jax.ShapeDtypeStruct((n_tokens, D), jnp.float32),
        mesh=mesh,
        compiler_params=pltpu.CompilerParams(use_tc_tiling_on_sc=False),
        scratch_types=[
            pltpu.VMEM((n_chunks, chunk), jnp.int32),
            pltpu.VMEM((n_chunks, chunk), jnp.int32),
            pltpu.VMEM((2, chunk, D), jnp.float32),
            pltpu.VMEM((2, chunk, D), jnp.float32),
            pltpu.VMEM((2, chunk, D), jnp.float32),
            pltpu.VMEM_SHARED((1000, D), jnp.float32),
            pltpu.SemaphoreType.DMA,
            pltpu.SemaphoreType.DMA,
            pltpu.SemaphoreType.DMA,
            pltpu.SemaphoreType.DMA,
            pltpu.SemaphoreType.DMA,
            pltpu.SemaphoreType.DMA,
        ],
    )
    def k(char_hbm, lang_hbm, ci_hbm, li_hbm, out_hbm,
          ci_all, li_all, buf_c, buf_l, buf_o, lang_v,
          gc0, gc1, gl0, gl1, ss0, ss1):
        gsem_c = [gc0, gc1]
        gsem_l = [gl0, gl1]
        ssem = [ss0, ss1]
        wid = lax.axis_index("s") * NC + lax.axis_index("c")
        cbase = wid * n_chunks
        tbase = cbase * chunk

        pltpu.sync_copy(ci_hbm.at[pl.ds(cbase, n_chunks)], ci_all)
        pltpu.sync_copy(li_hbm.at[pl.ds(cbase, n_chunks)], li_all)
        @pl.when(lax.axis_index("s") == 0)
        def _():
            pltpu.sync_copy(lang_hbm, lang_v)

        plsc.subcore_barrier()

        def start_gather(j, b):
            pltpu.async_copy(char_hbm.at[ci_all.at[j]], buf_c.at[b], gsem_c[b])
            pltpu.async_copy(lang_v.at[li_all.at[j]], buf_l.at[b], gsem_l[b])

        def wait_gather(j, b):
            pltpu.make_async_copy(char_hbm.at[ci_all.at[j]],
                                  buf_c.at[b], gsem_c[b]).wait()
            pltpu.make_async_copy(lang_v.at[li_all.at[j]],
                                  buf_l.at[b], gsem_l[b]).wait()

        def start_store(j, b):
            pltpu.async_copy(buf_o.at[b],
                             out_hbm.at[pl.ds(tbase + j * chunk, chunk)],
                             ssem[b])

        def wait_store(j, b):
            pltpu.make_async_copy(buf_o.at[b],
                                  out_hbm.at[pl.ds(tbase + j * chunk, chunk)],
                                  ssem[b]).wait()

        start_gather(0, 0)
        start_gather(1, 1)

        def body(g, carry):
            for b in range(2):
                j = g * 2 + b
                wait_gather(j, b)

                @pl.when(j >= 2)
                def _():
                    wait_store(j - 2, b)

                @plsc.parallel_loop(0, chunk, unroll=8)
                def add_row(r):
                    for p in range(D // LANES):
                        sl = pl.ds(p * LANES, LANES)
                        buf_o[b, r, sl] = buf_c[b, r, sl] + buf_l[b, r, sl]
                start_store(j, b)

                @pl.when(j + 2 < n_chunks)
                def _():
                    start_gather(j + 2, b)

            return carry

        lax.fori_loop(0, n_chunks // 2, body, 0)
        wait_store(n_chunks - 2, 0)
        wait_store(n_chunks - 1, 1)

    return k


def _fin_body(x_ref, i_ref, o_ref):
    cols, bb = o_ref.shape
    nq = cols // (2 * D)
    x3 = x_ref[...].reshape(bb, nq, 2 * D)
    ident = i_ref[...]
    for q in range(nq):
        oq = jax.lax.dot_general(
            x3[:, q, :], ident, (((0,), (0,)), ((), ())),
            preferred_element_type=jnp.float32)
        o_ref[pl.ds(q * 2 * D, 2 * D), :] = oq


def _finalize(out_pair, B, L):
    bb = 256
    cols = L * D
    rows = bb * L // 2
    return pl.pallas_call(
        _fin_body,
        grid=(B // bb,),
        in_specs=[pl.BlockSpec((rows, 2 * D), lambda i: (i, 0)),
                  pl.BlockSpec((bb, bb), lambda i: (0, 0))],
        out_specs=pl.BlockSpec((cols, bb), lambda i: (0, i)),
        out_shape=jax.ShapeDtypeStruct((cols, B), jnp.float32),
    )(out_pair, jnp.eye(bb, dtype=jnp.float32))


def kernel(char_indices, lang_indices, char_table, lang_table, W, b):
    B, L = char_indices.shape
    n = B * L
    char_proj, lang_proj = _project_tables(char_table, lang_table, W, b)
    chunk = 128
    ci = char_indices.reshape(n // chunk, chunk)
    li = lang_indices.reshape(n // chunk, chunk)
    gather_add = _make_gather_add(n, chunk=chunk)
    out = gather_add(char_proj, lang_proj, ci, li)
    out_t = _finalize(out.reshape(n // 2, 2 * D), B, L)
    return jnp.transpose(out_t.reshape(L, D, B), (2, 0, 1))


# speedup vs baseline: 1.8735x; 1.0054x over previous
import functools

import jax
import jax.numpy as jnp
from jax import lax
from jax.experimental import pallas as pl
from jax.experimental.pallas import tpu as pltpu
from jax.experimental.pallas import tpu_sc as plsc

D = 64
NC = 2
NS = 16
NW = NC * NS
LANES = 16


def _proj_body(x_ref, wt_ref, o_ref):
    o_ref[...] = jnp.dot(x_ref[...], wt_ref[...],
                         preferred_element_type=jnp.float32)


def _projb_body(x_ref, wt_ref, b_ref, o_ref):
    o_ref[...] = jnp.dot(x_ref[...], wt_ref[...],
                         preferred_element_type=jnp.float32) + b_ref[...]


def _project_tables(char_table, lang_table, W, b):
    V, _ = char_table.shape
    Vl, _ = lang_table.shape
    wt_c = W[:, :D].T
    wt_l = W[:, D:].T
    blk = 10000
    char_proj = pl.pallas_call(
        _proj_body,
        grid=(V // blk,),
        in_specs=[pl.BlockSpec((blk, D), lambda i: (i, 0)),
                  pl.BlockSpec((D, D), lambda i: (0, 0))],
        out_specs=pl.BlockSpec((blk, D), lambda i: (i, 0)),
        out_shape=jax.ShapeDtypeStruct((V, D), jnp.float32),
    )(char_table, wt_c)
    lang_proj = pl.pallas_call(
        _projb_body,
        in_specs=[pl.BlockSpec((Vl, D), lambda: (0, 0)),
                  pl.BlockSpec((D, D), lambda: (0, 0)),
                  pl.BlockSpec((1, D), lambda: (0, 0))],
        out_specs=pl.BlockSpec((Vl, D), lambda: (0, 0)),
        out_shape=jax.ShapeDtypeStruct((Vl, D), jnp.float32),
    )(lang_table, wt_l, b.reshape(1, D))
    return char_proj, lang_proj


def _make_gather_add(n_tokens, chunk):
    per_w = n_tokens // NW
    n_chunks = per_w // chunk
    mesh = plsc.VectorSubcoreMesh(core_axis_name="c", subcore_axis_name="s")

    @functools.partial(
        pl.kernel,
        out_type=jax.ShapeDtypeStruct((n_tokens, D), jnp.float32),
        mesh=mesh,
        compiler_params=pltpu.CompilerParams(use_tc_tiling_on_sc=False),
        scratch_types=[
            pltpu.VMEM((n_chunks, chunk), jnp.int32),
            pltpu.VMEM((n_chunks, chunk), jnp.int32),
            pltpu.VMEM((2, chunk, D), jnp.float32),
            pltpu.VMEM((2, chunk, D), jnp.float32),
            pltpu.VMEM((2, chunk, D), jnp.float32),
            pltpu.VMEM_SHARED((1000, D), jnp.float32),
            pltpu.SemaphoreType.DMA,
            pltpu.SemaphoreType.DMA,
            pltpu.SemaphoreType.DMA,
            pltpu.SemaphoreType.DMA,
            pltpu.SemaphoreType.DMA,
            pltpu.SemaphoreType.DMA,
        ],
    )
    def k(char_hbm, lang_hbm, ci_hbm, li_hbm, out_hbm,
          ci_all, li_all, buf_c, buf_l, buf_o, lang_v,
          gc0, gc1, gl0, gl1, ss0, ss1):
        gsem_c = [gc0, gc1]
        gsem_l = [gl0, gl1]
        ssem = [ss0, ss1]
        wid = lax.axis_index("s") * NC + lax.axis_index("c")
        cbase = wid * n_chunks
        tbase = cbase * chunk

        pltpu.sync_copy(ci_hbm.at[pl.ds(cbase, n_chunks)], ci_all)
        pltpu.sync_copy(li_hbm.at[pl.ds(cbase, n_chunks)], li_all)
        @pl.when(lax.axis_index("s") == 0)
        def _():
            pltpu.sync_copy(lang_hbm, lang_v)

        plsc.subcore_barrier()

        def start_gather(j, b):
            pltpu.async_copy(char_hbm.at[ci_all.at[j]], buf_c.at[b], gsem_c[b])
            pltpu.async_copy(lang_v.at[li_all.at[j]], buf_l.at[b], gsem_l[b])

        def wait_gather(j, b):
            pltpu.make_async_copy(char_hbm.at[ci_all.at[j]],
                                  buf_c.at[b], gsem_c[b]).wait()
            pltpu.make_async_copy(lang_v.at[li_all.at[j]],
                                  buf_l.at[b], gsem_l[b]).wait()

        def start_store(j, b):
            pltpu.async_copy(buf_o.at[b],
                             out_hbm.at[pl.ds(tbase + j * chunk, chunk)],
                             ssem[b])

        def wait_store(j, b):
            pltpu.make_async_copy(buf_o.at[b],
                                  out_hbm.at[pl.ds(tbase + j * chunk, chunk)],
                                  ssem[b]).wait()

        start_gather(0, 0)
        start_gather(1, 1)

        def body(g, carry):
            for b in range(2):
                j = g * 2 + b
                wait_gather(j, b)

                @pl.when(j >= 2)
                def _():
                    wait_store(j - 2, b)

                @plsc.parallel_loop(0, chunk, unroll=8)
                def add_row(r):
                    for p in range(D // LANES):
                        sl = pl.ds(p * LANES, LANES)
                        buf_o[b, r, sl] = buf_c[b, r, sl] + buf_l[b, r, sl]
                start_store(j, b)

                @pl.when(j + 2 < n_chunks)
                def _():
                    start_gather(j + 2, b)

            return carry

        lax.fori_loop(0, n_chunks // 2, body, 0)
        wait_store(n_chunks - 2, 0)
        wait_store(n_chunks - 1, 1)

    return k


def _fin_body(x_ref, i_ref, o_ref):
    cols, bb = o_ref.shape
    nq = cols // (2 * D)
    x3 = x_ref[...].reshape(bb, nq, 2 * D)
    ident = i_ref[...]
    for q in range(nq):
        oq = jax.lax.dot_general(
            x3[:, q, :], ident, (((0,), (0,)), ((), ())),
            preferred_element_type=jnp.float32)
        o_ref[pl.ds(q * 2 * D, 2 * D), :] = oq


def _finalize(out_pair, B, L):
    bb = 512
    cols = L * D
    rows = bb * L // 2
    return pl.pallas_call(
        _fin_body,
        grid=(B // bb,),
        in_specs=[pl.BlockSpec((rows, 2 * D), lambda i: (i, 0)),
                  pl.BlockSpec((bb, bb), lambda i: (0, 0))],
        out_specs=pl.BlockSpec((cols, bb), lambda i: (0, i)),
        out_shape=jax.ShapeDtypeStruct((cols, B), jnp.float32),
    )(out_pair, jnp.eye(bb, dtype=jnp.float32))


def kernel(char_indices, lang_indices, char_table, lang_table, W, b):
    B, L = char_indices.shape
    n = B * L
    char_proj, lang_proj = _project_tables(char_table, lang_table, W, b)
    chunk = 128
    ci = char_indices.reshape(n // chunk, chunk)
    li = lang_indices.reshape(n // chunk, chunk)
    gather_add = _make_gather_add(n, chunk=chunk)
    out = gather_add(char_proj, lang_proj, ci, li)
    out_t = _finalize(out.reshape(n // 2, 2 * D), B, L)
    return jnp.transpose(out_t.reshape(L, D, B), (2, 0, 1))
